# baseline (speedup 1.0000x reference)
import functools

import jax
import jax.numpy as jnp
from jax import lax
from jax.experimental import pallas as pl
from jax.experimental.pallas import tpu as pltpu
from jax.experimental.pallas import tpu_sc as plsc

FEATURE_SIZE = 1000000
FIELD_SIZE = 26
BATCH = 4096
MAX_LEN = 20

NUM_WORKERS = 32
BPW = BATCH // NUM_WORKERS
CHUNK = BPW * MAX_LEN
PER_FIELD = BATCH * MAX_LEN
OUT_PER_W = BPW * FIELD_SIZE
NGROUP = BPW // 16
PHASES = ((0, 7), (7, 14), (14, 20), (20, 26))
NSLOT = 7
TSLICE = 62504
TABLE_PAD = 16 * TSLICE


def _sc_kernel(vals_hbm, idx_hbm, seq_hbm, table_hbm, out_hbm,
               idxb, vb, wb, seq_buf, out_buf, table_sh,
               sem_i, sem_g, sem_v):
    info = plsc.get_sparse_core_info()
    nc = info.num_cores
    sid = lax.axis_index("s")
    wid = sid * nc + lax.axis_index("c")
    col0 = wid * CHUNK

    lane = lax.iota(jnp.int32, 16)
    lane20 = lane * MAX_LEN
    lane26 = lane * FIELD_SIZE

    @pl.when(sid == 0)
    def _():
        pltpu.sync_copy(table_hbm, table_sh)

    plsc.subcore_barrier()

    pltpu.sync_copy(seq_hbm.at[pl.ds(wid * OUT_PER_W, OUT_PER_W)], seq_buf)

    def fire_idx(p):
        lo, hi = PHASES[p]
        handles = []
        for j in range(hi - lo):
            src0 = (lo + j) * PER_FIELD + col0
            handles.append(pltpu.async_copy(
                idx_hbm.at[pl.ds(src0, CHUNK)],
                idxb.at[pl.ds(j * CHUNK, CHUNK)], sem_i))
        return handles

    def compute_fields(lo, hi):
        def field_body(f, c):
            base0 = (f - lo) * CHUNK

            def group_body(g, c2):
                acc = jnp.zeros((16,), jnp.float32)
                base = base0 + g * (16 * MAX_LEN)
                for l in range(MAX_LEN):
                    flat = base + l + lane20
                    acc = acc + (plsc.load_gather(wb, [flat])
                                 * plsc.load_gather(vb, [flat]))
                i_out = (g * 16) * FIELD_SIZE + lane26 + f
                sq = plsc.load_gather(seq_buf, [i_out]).astype(jnp.float32)
                plsc.store_scatter(out_buf, [i_out], acc / sq)
                return c2

            lax.fori_loop(0, NGROUP, group_body, 0)
            return c

        lax.fori_loop(lo, hi, field_body, 0)

    ih = fire_idx(0)
    for p, (lo, hi) in enumerate(PHASES):
        gh, vh = [], []
        for j in range(hi - lo):
            ih[j].wait()
            gh.append(pltpu.async_copy(
                table_sh.at[idxb.at[pl.ds(j * CHUNK, CHUNK)]],
                wb.at[pl.ds(j * CHUNK, CHUNK)], sem_g))
            src0 = (lo + j) * PER_FIELD + col0
            vh.append(pltpu.async_copy(
                vals_hbm.at[pl.ds(src0, CHUNK)],
                vb.at[pl.ds(j * CHUNK, CHUNK)], sem_v))
        for h in gh:
            h.wait()
        if p + 1 < len(PHASES):
            ih = fire_idx(p + 1)
        for h in vh:
            h.wait()
        compute_fields(lo, hi)

    pltpu.sync_copy(out_buf, out_hbm.at[pl.ds(wid * OUT_PER_W, OUT_PER_W)])


@jax.jit
def _first_order(vals_flat, idx_flat, seq_flat, table_flat):
    mesh = plsc.VectorSubcoreMesh(core_axis_name="c", subcore_axis_name="s")
    run = functools.partial(
        pl.kernel,
        out_type=jax.ShapeDtypeStruct((BATCH * FIELD_SIZE,), jnp.float32),
        mesh=mesh,
        compiler_params=pltpu.CompilerParams(needs_layout_passes=False),
        scratch_types=[
            pltpu.VMEM((NSLOT * CHUNK,), jnp.int32),
            pltpu.VMEM((NSLOT * CHUNK,), jnp.float32),
            pltpu.VMEM((NSLOT * CHUNK,), jnp.float32),
            pltpu.VMEM((OUT_PER_W,), jnp.int32),
            pltpu.VMEM((OUT_PER_W,), jnp.float32),
            pltpu.VMEM_SHARED((TABLE_PAD,), jnp.float32),
            pltpu.SemaphoreType.DMA,
            pltpu.SemaphoreType.DMA,
            pltpu.SemaphoreType.DMA,
        ],
    )(_sc_kernel)
    return run(vals_flat, idx_flat, seq_flat, table_flat)


def kernel(feature_values, feature_idx, seq_lens, weights_first_order):
    vals_flat = feature_values.reshape(FIELD_SIZE * PER_FIELD)
    idx_flat = feature_idx.astype(jnp.int32).reshape(FIELD_SIZE * PER_FIELD)
    seq_flat = seq_lens.reshape(BATCH * FIELD_SIZE)
    table_flat = jnp.concatenate([
        weights_first_order.reshape(FEATURE_SIZE + 2),
        jnp.zeros((TABLE_PAD - (FEATURE_SIZE + 2),), jnp.float32),
    ])
    out = _first_order(vals_flat, idx_flat, seq_flat, table_flat)
    return out.reshape(BATCH, FIELD_SIZE)
